# baseline (speedup 1.0000x reference)
import functools

import jax
import jax.numpy as jnp
from jax import lax
from jax.experimental import pallas as pl
from jax.experimental.pallas import tpu as pltpu
from jax.experimental.pallas import tpu_sc as plsc

N = 10000
D = 128
NB = 3
E = 320000
EPS = 1e-5
NUM_USERS = 5000

QTR = N // 4
STRIDE = 2560
SP_ROWS = NB * STRIDE
NSC = 2
NTILE = 16
CH = 256
ROWS_PER_TILE = SP_ROWS // NTILE
ROWS_PER_TILE_B = STRIDE // NTILE
ZROWS = 96
CBUF = 1280


def _sc_edge_pass(tstride, with_cnt, qpass, table_hbm, src_hbm, dst_hbm,
                  typ_hbm, zeros_hbm, agg_hbm, cnt_hbm, agg_sp, cnt_sp,
                  rows_v, gixa, sixa, gixb, sixb, src_st, dst_st, typ_st,
                  ones_v, zero_v, cbuf_v, sem_ia, sem_ib, sem_g, sem_ca,
                  sem_cb):
    c = lax.axis_index("c")
    s = lax.axis_index("s")
    base = (2 * qpass + c) * QTR
    ep = src_hbm.shape[0] // NTILE
    tbase = s * ep
    npair = ep // (2 * CH)

    pltpu.sync_copy(zeros_hbm, zero_v)
    ones16 = jnp.full((16,), 1.0, dtype=jnp.float32)
    zeros16 = jnp.zeros((16,), dtype=jnp.float32)
    for j in range(CH // 16):
        ones_v[pl.ds(j * 16, 16)] = ones16
    for j in range(CBUF // 16):
        cbuf_v[pl.ds(j * 16, 16)] = zeros16
    zrow = s * ROWS_PER_TILE
    for z in range(ROWS_PER_TILE // ZROWS):
        pltpu.sync_copy(zero_v, agg_sp.at[pl.ds(zrow + z * ZROWS, ZROWS)])
    if with_cnt:
        pltpu.sync_copy(cbuf_v.at[pl.ds(0, ROWS_PER_TILE)],
                        cnt_sp.at[pl.ds(zrow, ROWS_PER_TILE)])
    plsc.subcore_barrier()

    def issue_inputs(g, st_set, sem):
        off = pl.multiple_of(tbase + g * CH, CH)
        pltpu.async_copy(src_hbm.at[pl.ds(off, CH)], src_st.at[st_set], sem)
        pltpu.async_copy(dst_hbm.at[pl.ds(off, CH)], dst_st.at[st_set], sem)
        pltpu.async_copy(typ_hbm.at[pl.ds(off, CH)], typ_st.at[st_set], sem)

    def wait_inputs(st_set, sem):
        for ref in (src_st, dst_st, typ_st):
            pltpu.make_async_copy(src_hbm.at[pl.ds(0, CH)], ref.at[st_set],
                                  sem).wait()

    def compute_idx(st_set, gix, six):
        for j in range(CH // 16):
            sl = pl.ds(j * 16, 16)
            sv = src_st[st_set, sl]
            dv = dst_st[st_set, sl]
            tv = typ_st[st_set, sl]
            gi = sv + tv * tstride if tstride else sv
            local = dv - base
            inb = (local >= 0) & (local < QTR)
            si = tv * STRIDE + jnp.where(inb, local, QTR + s)
            gix[pl.ds(j * 16, 16)] = gi
            six[pl.ds(j * 16, 16)] = si

    def do_chunk(st_set, gix, six, semc):
        compute_idx(st_set, gix, six)
        pltpu.async_copy(table_hbm.at[gix], rows_v, sem_g).wait()
        pltpu.sync_copy(rows_v, agg_sp.at[six], add=True)
        if with_cnt:
            pltpu.async_copy(ones_v, cnt_sp.at[six], semc, add=True)

    def wait_cnt(six, semc):
        if with_cnt:
            pltpu.make_async_copy(ones_v, cnt_sp.at[six], semc).wait()

    issue_inputs(0, 0, sem_ia)

    def pair_body(k, carry):
        wait_inputs(0, sem_ia)
        issue_inputs(2 * k + 1, 1, sem_ib)

        @pl.when(k >= 1)
        def _():
            wait_cnt(sixa, sem_ca)
        do_chunk(0, gixa, sixa, sem_ca)
        wait_inputs(1, sem_ib)

        @pl.when(k + 1 < npair)
        def _():
            issue_inputs(2 * k + 2, 0, sem_ia)

        @pl.when(k >= 1)
        def _():
            wait_cnt(sixb, sem_cb)
        do_chunk(1, gixb, sixb, sem_cb)
        return carry

    lax.fori_loop(0, npair, pair_body, 0)
    wait_cnt(sixa, sem_ca)
    wait_cnt(sixb, sem_cb)
    plsc.subcore_barrier()

    for b in range(NB):
        r0 = b * STRIDE + s * ROWS_PER_TILE_B
        pltpu.sync_copy(agg_sp.at[pl.ds(r0, ROWS_PER_TILE_B)],
                        agg_hbm.at[b, c, pl.ds(s * ROWS_PER_TILE_B,
                                               ROWS_PER_TILE_B)])
    if with_cnt:
        @pl.when(s == 0)
        def _():
            off = pl.multiple_of(c * SP_ROWS, SP_ROWS)
            for k in range(SP_ROWS // CBUF):
                pltpu.sync_copy(cnt_sp.at[pl.ds(k * CBUF, CBUF)], cbuf_v)
                pltpu.sync_copy(cbuf_v, cnt_hbm.at[pl.ds(off + k * CBUF,
                                                         CBUF)])


def _make_sc_pass(tstride, with_cnt, qpass):
    mesh = plsc.VectorSubcoreMesh(core_axis_name="c", subcore_axis_name="s",
                                  num_cores=NSC, num_subcores=NTILE)
    return pl.kernel(
        functools.partial(_sc_edge_pass, tstride, with_cnt, qpass),
        out_type=(
            jax.ShapeDtypeStruct((NB, NSC, STRIDE, D), jnp.float32),
            jax.ShapeDtypeStruct((NSC * SP_ROWS,), jnp.float32),
        ),
        mesh=mesh,
        scratch_types=[
            pltpu.VMEM_SHARED((SP_ROWS, D), jnp.float32),
            pltpu.VMEM_SHARED((SP_ROWS,), jnp.float32),
            pltpu.VMEM((CH, D), jnp.float32),
            pltpu.VMEM((CH,), jnp.int32),
            pltpu.VMEM((CH,), jnp.int32),
            pltpu.VMEM((CH,), jnp.int32),
            pltpu.VMEM((CH,), jnp.int32),
            pltpu.VMEM((2, CH), jnp.int32),
            pltpu.VMEM((2, CH), jnp.int32),
            pltpu.VMEM((2, CH), jnp.int32),
            pltpu.VMEM((CH,), jnp.float32),
            pltpu.VMEM((ZROWS, D), jnp.float32),
            pltpu.VMEM((CBUF,), jnp.float32),
            pltpu.SemaphoreType.DMA,
            pltpu.SemaphoreType.DMA,
            pltpu.SemaphoreType.DMA,
            pltpu.SemaphoreType.DMA,
            pltpu.SemaphoreType.DMA,
        ],
        name=f"sc_edge_pass_l{1 if tstride else 0}q{qpass}",
    )


def _dense0_body(ue_ref, itf_ref, wu_ref, bu_ref, wi_ref, bi_ref, x_ref):
    xu = jnp.dot(ue_ref[...], wu_ref[...].T,
                 preferred_element_type=jnp.float32) + bu_ref[...]
    xi = jnp.dot(itf_ref[...], wi_ref[...].T,
                 preferred_element_type=jnp.float32) + bi_ref[...]
    x_ref[0:NUM_USERS, :] = xu
    x_ref[NUM_USERS:N, :] = xi


def _layer_body(relu, residual, hin_ref, x_ref, agga_ref, aggb_ref, cnt_ref,
                wr_ref, wn_ref, bias_ref, gamma_ref, beta_ref, out_ref):
    inv = 1.0 / jnp.maximum(cnt_ref[0, 0], 1.0)
    agg = jnp.concatenate([agga_ref[0, 0, :QTR, :], agga_ref[0, 1, :QTR, :],
                           aggb_ref[0, 0, :QTR, :], aggb_ref[0, 1, :QTR, :]],
                          axis=0) * inv[:, None]
    hin = hin_ref[...].reshape(N, D)
    pre = (jnp.dot(hin, wr_ref[0].T, preferred_element_type=jnp.float32)
           + jnp.dot(agg, wn_ref[0].T, preferred_element_type=jnp.float32)
           + bias_ref[0])
    mean = jnp.mean(pre, axis=0)
    var = jnp.mean((pre - mean) ** 2, axis=0)
    h = (pre - mean) / jnp.sqrt(var + EPS) * gamma_ref[0] + beta_ref[0]
    if relu:
        h = jnp.maximum(h, 0.0)
    if residual:
        h = x_ref[...] + h
    out_ref[...] = h.reshape(out_ref.shape)


def _fusion_body(x_ref, stack_ref, qw_ref, qb_ref, kw_ref, kb_ref, fw_ref,
                 fb_ref, rw_ref, rb_ref, out_ref):
    x = x_ref[...]
    q = jnp.dot(x, qw_ref[...].T, preferred_element_type=jnp.float32) \
        + qb_ref[...]
    logits = []
    for b in range(NB):
        kb = jnp.dot(stack_ref[b], kw_ref[b].T,
                     preferred_element_type=jnp.float32) + kb_ref[b]
        logits.append(jnp.sum(q * kb, axis=-1))
    lg = jnp.stack(logits, axis=-1)
    lg = lg - jnp.max(lg, axis=-1, keepdims=True)
    ex = jnp.exp(lg)
    attn = ex / jnp.sum(ex, axis=-1, keepdims=True)
    fused = (attn[:, 0:1] * stack_ref[0] + attn[:, 1:2] * stack_ref[1]
             + attn[:, 2:3] * stack_ref[2])
    fused = jnp.dot(fused, fw_ref[...].T,
                    preferred_element_type=jnp.float32) + fb_ref[...]
    out = jnp.dot(fused, rw_ref[...].T,
                  preferred_element_type=jnp.float32) + rb_ref[...]
    out_ref[...] = jnp.maximum(out, 0.0)


def _layer_call(relu, residual, shared_hin):
    hin_map = (lambda b: (0, 0, 0)) if shared_hin else (lambda b: (b, 0, 0))
    return pl.pallas_call(
        functools.partial(_layer_body, relu, residual),
        grid=(NB,),
        in_specs=[
            pl.BlockSpec((1, N, D), hin_map),
            pl.BlockSpec((N, D), lambda b: (0, 0)),
            pl.BlockSpec((1, NSC, STRIDE, D), lambda b: (b, 0, 0, 0)),
            pl.BlockSpec((1, NSC, STRIDE, D), lambda b: (b, 0, 0, 0)),
            pl.BlockSpec((1, 1, N), lambda b: (b, 0, 0)),
            pl.BlockSpec((1, D, D), lambda b: (b, 0, 0)),
            pl.BlockSpec((1, D, D), lambda b: (b, 0, 0)),
            pl.BlockSpec((1, 1, D), lambda b: (b, 0, 0)),
            pl.BlockSpec((1, 1, D), lambda b: (b, 0, 0)),
            pl.BlockSpec((1, 1, D), lambda b: (b, 0, 0)),
        ],
        out_specs=pl.BlockSpec((1, N, D), lambda b: (b, 0, 0)),
        out_shape=jax.ShapeDtypeStruct((NB, N, D), jnp.float32),
    )


def kernel(edge_index, edge_type, node_ids, item_feats, user_emb,
           user_proj_W, user_proj_b, item_proj_W, item_proj_b,
           sage_Wroot, sage_Wneigh, sage_bias, bn_gamma, bn_beta,
           query_W, query_b, key_W, key_b, fuse_W, fuse_b,
           refine_W, refine_b):
    del node_ids

    ep = ((E + NTILE * 2 * CH - 1) // (NTILE * 2 * CH)) * 2 * CH
    e_pad = ep * NTILE
    pad = e_pad - E
    src = jnp.concatenate([edge_index[0].astype(jnp.int32),
                           jnp.zeros((pad,), jnp.int32)])
    dst = jnp.concatenate([edge_index[1].astype(jnp.int32),
                           jnp.full((pad,), -1, jnp.int32)])
    typ = jnp.concatenate([edge_type.astype(jnp.int32),
                           jnp.zeros((pad,), jnp.int32)])
    zeros_hbm = jnp.zeros((ZROWS, D), jnp.float32)

    x = pl.pallas_call(
        _dense0_body,
        out_shape=jax.ShapeDtypeStruct((N, D), jnp.float32),
    )(user_emb, item_feats, user_proj_W, user_proj_b[None], item_proj_W,
      item_proj_b[None])

    agg0a, cnta = _make_sc_pass(0, True, 0)(x, src, dst, typ, zeros_hbm)
    agg0b, cntb = _make_sc_pass(0, True, 1)(x, src, dst, typ, zeros_hbm)
    ca = cnta.reshape(NSC, NB, STRIDE)[:, :, :QTR]
    cb = cntb.reshape(NSC, NB, STRIDE)[:, :, :QTR]
    cnt = jnp.concatenate([ca.transpose(1, 0, 2), cb.transpose(1, 0, 2)],
                          axis=1).reshape(NB, 1, N)

    h0 = _layer_call(True, False, True)(
        x[None], x, agg0a, agg0b, cnt,
        sage_Wroot[:, 0], sage_Wneigh[:, 0], sage_bias[:, 0][:, None, :],
        bn_gamma[:, 0][:, None, :], bn_beta[:, 0][:, None, :])
    h0_tab = h0.reshape(NB * N, D)

    agg1a, _ = _make_sc_pass(N, False, 0)(h0_tab, src, dst, typ, zeros_hbm)
    agg1b, _ = _make_sc_pass(N, False, 1)(h0_tab, src, dst, typ, zeros_hbm)

    stack = _layer_call(False, True, False)(
        h0, x, agg1a, agg1b, cnt, sage_Wroot[:, 1], sage_Wneigh[:, 1],
        sage_bias[:, 1][:, None, :], bn_gamma[:, 1][:, None, :],
        bn_beta[:, 1][:, None, :])

    BLK = 2000
    final = pl.pallas_call(
        _fusion_body,
        grid=(N // BLK,),
        in_specs=[
            pl.BlockSpec((BLK, D), lambda i: (i, 0)),
            pl.BlockSpec((NB, BLK, D), lambda i: (0, i, 0)),
            pl.BlockSpec((D, D), lambda i: (0, 0)),
            pl.BlockSpec((1, D), lambda i: (0, 0)),
            pl.BlockSpec((NB, D, D), lambda i: (0, 0, 0)),
            pl.BlockSpec((NB, D), lambda i: (0, 0)),
            pl.BlockSpec((D, D), lambda i: (0, 0)),
            pl.BlockSpec((1, D), lambda i: (0, 0)),
            pl.BlockSpec((D, D), lambda i: (0, 0)),
            pl.BlockSpec((1, D), lambda i: (0, 0)),
        ],
        out_specs=pl.BlockSpec((BLK, D), lambda i: (i, 0)),
        out_shape=jax.ShapeDtypeStruct((N, D), jnp.float32),
    )(x, stack, query_W, query_b[None], key_W, key_b, fuse_W, fuse_b[None],
      refine_W, refine_b[None])
    return final


# speedup vs baseline: 1.4184x; 1.4184x over previous
import functools

import jax
import jax.numpy as jnp
from jax import lax
from jax.experimental import pallas as pl
from jax.experimental.pallas import tpu as pltpu
from jax.experimental.pallas import tpu_sc as plsc

N = 10000
D = 128
NB = 3
E = 320000
EPS = 1e-5
NUM_USERS = 5000

QTR = N // 4
STRIDE = 2560
SP_ROWS = NB * STRIDE
NSC = 2
NTILE = 16
CH = 128
ROWS_PER_TILE = SP_ROWS // NTILE
ROWS_PER_TILE_B = STRIDE // NTILE
ZROWS = 96
CBUF = 1280


def _sc_edge_pass(tstride, with_cnt, qpass, table_hbm, src_hbm, dst_hbm,
                  typ_hbm, zeros_hbm, agg_hbm, cnt_hbm, agg_sp, cnt_sp,
                  rows_v, gixa, sixa, gixb, sixb, src_st, dst_st, typ_st,
                  ones_v, zero_v, cbuf_v, sem_ia, sem_ib, sem_g, sem_ca,
                  sem_cb):
    c = lax.axis_index("c")
    s = lax.axis_index("s")
    base = (2 * qpass + c) * QTR
    ep = src_hbm.shape[0] // NTILE
    tbase = s * ep
    npair = ep // (2 * CH)

    pltpu.sync_copy(zeros_hbm, zero_v)
    ones16 = jnp.full((16,), 1.0, dtype=jnp.float32)
    zeros16 = jnp.zeros((16,), dtype=jnp.float32)
    for j in range(CH // 16):
        ones_v[pl.ds(j * 16, 16)] = ones16
    for j in range(CBUF // 16):
        cbuf_v[pl.ds(j * 16, 16)] = zeros16
    zrow = s * ROWS_PER_TILE
    for z in range(ROWS_PER_TILE // ZROWS):
        pltpu.sync_copy(zero_v, agg_sp.at[pl.ds(zrow + z * ZROWS, ZROWS)])
    if with_cnt:
        pltpu.sync_copy(cbuf_v.at[pl.ds(0, ROWS_PER_TILE)],
                        cnt_sp.at[pl.ds(zrow, ROWS_PER_TILE)])
    plsc.subcore_barrier()

    def issue_inputs(g, st_set, sem):
        off = pl.multiple_of(tbase + g * CH, CH)
        pltpu.async_copy(src_hbm.at[pl.ds(off, CH)], src_st.at[st_set], sem)
        pltpu.async_copy(dst_hbm.at[pl.ds(off, CH)], dst_st.at[st_set], sem)
        pltpu.async_copy(typ_hbm.at[pl.ds(off, CH)], typ_st.at[st_set], sem)

    def wait_inputs(st_set, sem):
        for ref in (src_st, dst_st, typ_st):
            pltpu.make_async_copy(src_hbm.at[pl.ds(0, CH)], ref.at[st_set],
                                  sem).wait()

    def compute_idx(st_set, gix, six):
        for j in range(CH // 16):
            sl = pl.ds(j * 16, 16)
            sv = src_st[st_set, sl]
            dv = dst_st[st_set, sl]
            tv = typ_st[st_set, sl]
            gi = sv + tv * tstride if tstride else sv
            local = dv - base
            inb = (local >= 0) & (local < QTR)
            si = tv * STRIDE + jnp.where(inb, local, QTR + s)
            gix[pl.ds(j * 16, 16)] = gi
            six[pl.ds(j * 16, 16)] = si

    def do_chunk(st_set, gix, six, semc):
        compute_idx(st_set, gix, six)
        pltpu.async_copy(table_hbm.at[gix], rows_v, sem_g).wait()
        pltpu.sync_copy(rows_v, agg_sp.at[six], add=True)
        if with_cnt:
            pltpu.async_copy(ones_v, cnt_sp.at[six], semc, add=True)

    def wait_cnt(six, semc):
        if with_cnt:
            pltpu.make_async_copy(ones_v, cnt_sp.at[six], semc).wait()

    issue_inputs(0, 0, sem_ia)

    def pair_body(k, carry):
        wait_inputs(0, sem_ia)
        issue_inputs(2 * k + 1, 1, sem_ib)

        @pl.when(k >= 1)
        def _():
            wait_cnt(sixa, sem_ca)
        do_chunk(0, gixa, sixa, sem_ca)
        wait_inputs(1, sem_ib)

        @pl.when(k + 1 < npair)
        def _():
            issue_inputs(2 * k + 2, 0, sem_ia)

        @pl.when(k >= 1)
        def _():
            wait_cnt(sixb, sem_cb)
        do_chunk(1, gixb, sixb, sem_cb)
        return carry

    lax.fori_loop(0, npair, pair_body, 0)
    wait_cnt(sixa, sem_ca)
    wait_cnt(sixb, sem_cb)
    plsc.subcore_barrier()

    for b in range(NB):
        r0 = b * STRIDE + s * ROWS_PER_TILE_B
        pltpu.sync_copy(agg_sp.at[pl.ds(r0, ROWS_PER_TILE_B)],
                        agg_hbm.at[b, c, pl.ds(s * ROWS_PER_TILE_B,
                                               ROWS_PER_TILE_B)])
    if with_cnt:
        @pl.when(s == 0)
        def _():
            off = pl.multiple_of(c * SP_ROWS, SP_ROWS)
            for k in range(SP_ROWS // CBUF):
                pltpu.sync_copy(cnt_sp.at[pl.ds(k * CBUF, CBUF)], cbuf_v)
                pltpu.sync_copy(cbuf_v, cnt_hbm.at[pl.ds(off + k * CBUF,
                                                         CBUF)])


def _make_sc_pass(tstride, with_cnt, qpass):
    mesh = plsc.VectorSubcoreMesh(core_axis_name="c", subcore_axis_name="s",
                                  num_cores=NSC, num_subcores=NTILE)
    return pl.kernel(
        functools.partial(_sc_edge_pass, tstride, with_cnt, qpass),
        out_type=(
            jax.ShapeDtypeStruct((NB, NSC, STRIDE, D), jnp.float32),
            jax.ShapeDtypeStruct((NSC * SP_ROWS,), jnp.float32),
        ),
        mesh=mesh,
        scratch_types=[
            pltpu.VMEM_SHARED((SP_ROWS, D), jnp.float32),
            pltpu.VMEM_SHARED((SP_ROWS,), jnp.float32),
            pltpu.VMEM((CH, D), jnp.float32),
            pltpu.VMEM((CH,), jnp.int32),
            pltpu.VMEM((CH,), jnp.int32),
            pltpu.VMEM((CH,), jnp.int32),
            pltpu.VMEM((CH,), jnp.int32),
            pltpu.VMEM((2, CH), jnp.int32),
            pltpu.VMEM((2, CH), jnp.int32),
            pltpu.VMEM((2, CH), jnp.int32),
            pltpu.VMEM((CH,), jnp.float32),
            pltpu.VMEM((ZROWS, D), jnp.float32),
            pltpu.VMEM((CBUF,), jnp.float32),
            pltpu.SemaphoreType.DMA,
            pltpu.SemaphoreType.DMA,
            pltpu.SemaphoreType.DMA,
            pltpu.SemaphoreType.DMA,
            pltpu.SemaphoreType.DMA,
        ],
        name=f"sc_edge_pass_l{1 if tstride else 0}q{qpass}",
    )


def _dense0_body(ue_ref, itf_ref, wu_ref, bu_ref, wi_ref, bi_ref, x_ref):
    xu = jnp.dot(ue_ref[...], wu_ref[...].T,
                 preferred_element_type=jnp.float32) + bu_ref[...]
    xi = jnp.dot(itf_ref[...], wi_ref[...].T,
                 preferred_element_type=jnp.float32) + bi_ref[...]
    x_ref[0:NUM_USERS, :] = xu
    x_ref[NUM_USERS:N, :] = xi


def _layer_body(relu, residual, hin_ref, x_ref, agga_ref, aggb_ref, cnt_ref,
                wr_ref, wn_ref, bias_ref, gamma_ref, beta_ref, out_ref):
    inv = 1.0 / jnp.maximum(cnt_ref[0, 0], 1.0)
    agg = jnp.concatenate([agga_ref[0, 0, :QTR, :], agga_ref[0, 1, :QTR, :],
                           aggb_ref[0, 0, :QTR, :], aggb_ref[0, 1, :QTR, :]],
                          axis=0) * inv[:, None]
    hin = hin_ref[...].reshape(N, D)
    pre = (jnp.dot(hin, wr_ref[0].T, preferred_element_type=jnp.float32)
           + jnp.dot(agg, wn_ref[0].T, preferred_element_type=jnp.float32)
           + bias_ref[0])
    mean = jnp.mean(pre, axis=0)
    var = jnp.mean((pre - mean) ** 2, axis=0)
    h = (pre - mean) / jnp.sqrt(var + EPS) * gamma_ref[0] + beta_ref[0]
    if relu:
        h = jnp.maximum(h, 0.0)
    if residual:
        h = x_ref[...] + h
    out_ref[...] = h.reshape(out_ref.shape)


def _fusion_body(x_ref, stack_ref, qw_ref, qb_ref, kw_ref, kb_ref, fw_ref,
                 fb_ref, rw_ref, rb_ref, out_ref):
    x = x_ref[...]
    q = jnp.dot(x, qw_ref[...].T, preferred_element_type=jnp.float32) \
        + qb_ref[...]
    logits = []
    for b in range(NB):
        kb = jnp.dot(stack_ref[b], kw_ref[b].T,
                     preferred_element_type=jnp.float32) + kb_ref[b]
        logits.append(jnp.sum(q * kb, axis=-1))
    lg = jnp.stack(logits, axis=-1)
    lg = lg - jnp.max(lg, axis=-1, keepdims=True)
    ex = jnp.exp(lg)
    attn = ex / jnp.sum(ex, axis=-1, keepdims=True)
    fused = (attn[:, 0:1] * stack_ref[0] + attn[:, 1:2] * stack_ref[1]
             + attn[:, 2:3] * stack_ref[2])
    fused = jnp.dot(fused, fw_ref[...].T,
                    preferred_element_type=jnp.float32) + fb_ref[...]
    out = jnp.dot(fused, rw_ref[...].T,
                  preferred_element_type=jnp.float32) + rb_ref[...]
    out_ref[...] = jnp.maximum(out, 0.0)


def _layer_call(relu, residual, shared_hin):
    hin_map = (lambda b: (0, 0, 0)) if shared_hin else (lambda b: (b, 0, 0))
    return pl.pallas_call(
        functools.partial(_layer_body, relu, residual),
        grid=(NB,),
        in_specs=[
            pl.BlockSpec((1, N, D), hin_map),
            pl.BlockSpec((N, D), lambda b: (0, 0)),
            pl.BlockSpec((1, NSC, STRIDE, D), lambda b: (b, 0, 0, 0)),
            pl.BlockSpec((1, NSC, STRIDE, D), lambda b: (b, 0, 0, 0)),
            pl.BlockSpec((1, 1, N), lambda b: (b, 0, 0)),
            pl.BlockSpec((1, D, D), lambda b: (b, 0, 0)),
            pl.BlockSpec((1, D, D), lambda b: (b, 0, 0)),
            pl.BlockSpec((1, 1, D), lambda b: (b, 0, 0)),
            pl.BlockSpec((1, 1, D), lambda b: (b, 0, 0)),
            pl.BlockSpec((1, 1, D), lambda b: (b, 0, 0)),
        ],
        out_specs=pl.BlockSpec((1, N, D), lambda b: (b, 0, 0)),
        out_shape=jax.ShapeDtypeStruct((NB, N, D), jnp.float32),
    )


def kernel(edge_index, edge_type, node_ids, item_feats, user_emb,
           user_proj_W, user_proj_b, item_proj_W, item_proj_b,
           sage_Wroot, sage_Wneigh, sage_bias, bn_gamma, bn_beta,
           query_W, query_b, key_W, key_b, fuse_W, fuse_b,
           refine_W, refine_b):
    del node_ids

    ep = ((E + NTILE * 2 * CH - 1) // (NTILE * 2 * CH)) * 2 * CH
    e_pad = ep * NTILE
    pad = e_pad - E
    src = jnp.concatenate([edge_index[0].astype(jnp.int32),
                           jnp.zeros((pad,), jnp.int32)])
    dst = jnp.concatenate([edge_index[1].astype(jnp.int32),
                           jnp.full((pad,), -1, jnp.int32)])
    typ = jnp.concatenate([edge_type.astype(jnp.int32),
                           jnp.zeros((pad,), jnp.int32)])
    zeros_hbm = jnp.zeros((ZROWS, D), jnp.float32)

    x = pl.pallas_call(
        _dense0_body,
        out_shape=jax.ShapeDtypeStruct((N, D), jnp.float32),
    )(user_emb, item_feats, user_proj_W, user_proj_b[None], item_proj_W,
      item_proj_b[None])

    agg0a, cnta = _make_sc_pass(0, True, 0)(x, src, dst, typ, zeros_hbm)
    agg0b, cntb = _make_sc_pass(0, True, 1)(x, src, dst, typ, zeros_hbm)
    ca = cnta.reshape(NSC, NB, STRIDE)[:, :, :QTR]
    cb = cntb.reshape(NSC, NB, STRIDE)[:, :, :QTR]
    cnt = jnp.concatenate([ca.transpose(1, 0, 2), cb.transpose(1, 0, 2)],
                          axis=1).reshape(NB, 1, N)

    h0 = _layer_call(True, False, True)(
        x[None], x, agg0a, agg0b, cnt,
        sage_Wroot[:, 0], sage_Wneigh[:, 0], sage_bias[:, 0][:, None, :],
        bn_gamma[:, 0][:, None, :], bn_beta[:, 0][:, None, :])
    h0_tab = h0.reshape(NB * N, D)

    agg1a, _ = _make_sc_pass(N, False, 0)(h0_tab, src, dst, typ, zeros_hbm)
    agg1b, _ = _make_sc_pass(N, False, 1)(h0_tab, src, dst, typ, zeros_hbm)

    stack = _layer_call(False, True, False)(
        h0, x, agg1a, agg1b, cnt, sage_Wroot[:, 1], sage_Wneigh[:, 1],
        sage_bias[:, 1][:, None, :], bn_gamma[:, 1][:, None, :],
        bn_beta[:, 1][:, None, :])

    BLK = 2000
    final = pl.pallas_call(
        _fusion_body,
        grid=(N // BLK,),
        in_specs=[
            pl.BlockSpec((BLK, D), lambda i: (i, 0)),
            pl.BlockSpec((NB, BLK, D), lambda i: (0, i, 0)),
            pl.BlockSpec((D, D), lambda i: (0, 0)),
            pl.BlockSpec((1, D), lambda i: (0, 0)),
            pl.BlockSpec((NB, D, D), lambda i: (0, 0, 0)),
            pl.BlockSpec((NB, D), lambda i: (0, 0)),
            pl.BlockSpec((D, D), lambda i: (0, 0)),
            pl.BlockSpec((1, D), lambda i: (0, 0)),
            pl.BlockSpec((D, D), lambda i: (0, 0)),
            pl.BlockSpec((1, D), lambda i: (0, 0)),
        ],
        out_specs=pl.BlockSpec((BLK, D), lambda i: (i, 0)),
        out_shape=jax.ShapeDtypeStruct((N, D), jnp.float32),
    )(x, stack, query_W, query_b[None], key_W, key_b, fuse_W, fuse_b[None],
      refine_W, refine_b[None])
    return final


# speedup vs baseline: 1.5630x; 1.1019x over previous
import functools

import jax
import jax.numpy as jnp
from jax import lax
from jax.experimental import pallas as pl
from jax.experimental.pallas import tpu as pltpu
from jax.experimental.pallas import tpu_sc as plsc

N = 10000
D = 128
NB = 3
E = 320000
EPS = 1e-5
NUM_USERS = 5000

QTR = N // 4
STRIDE = 2560
SP_ROWS = NB * STRIDE
NSC = 2
NTILE = 16
CH = 128
ROWS_PER_TILE = SP_ROWS // NTILE
ROWS_PER_TILE_B = STRIDE // NTILE
ZROWS = 96
CBUF = 1280


def _sc_edge_pass(tstride, with_cnt, qpass, table_hbm, src_hbm, dst_hbm,
                  typ_hbm, zeros_hbm, agg_hbm, cnt_hbm, agg_sp, cnt_sp,
                  rows_v, gixa, sixa, gixb, sixb, src_st, dst_st, typ_st,
                  ones_v, zero_v, cbuf_v, sem_ia, sem_ib, sem_g, sem_gb,
                  sem_ca, sem_cb):
    c = lax.axis_index("c")
    s = lax.axis_index("s")
    base = (2 * qpass + c) * QTR
    ep = src_hbm.shape[0] // NTILE
    tbase = s * ep
    npair = ep // (2 * CH)

    pltpu.sync_copy(zeros_hbm, zero_v)
    ones16 = jnp.full((16,), 1.0, dtype=jnp.float32)
    zeros16 = jnp.zeros((16,), dtype=jnp.float32)
    for j in range(CH // 16):
        ones_v[pl.ds(j * 16, 16)] = ones16
    for j in range(CBUF // 16):
        cbuf_v[pl.ds(j * 16, 16)] = zeros16
    zrow = s * ROWS_PER_TILE
    for z in range(ROWS_PER_TILE // ZROWS):
        pltpu.sync_copy(zero_v, agg_sp.at[pl.ds(zrow + z * ZROWS, ZROWS)])
    if with_cnt:
        pltpu.sync_copy(cbuf_v.at[pl.ds(0, ROWS_PER_TILE)],
                        cnt_sp.at[pl.ds(zrow, ROWS_PER_TILE)])
    plsc.subcore_barrier()

    def issue_inputs(g, st_set, sem):
        off = pl.multiple_of(tbase + g * CH, CH)
        pltpu.async_copy(src_hbm.at[pl.ds(off, CH)], src_st.at[st_set], sem)
        pltpu.async_copy(dst_hbm.at[pl.ds(off, CH)], dst_st.at[st_set], sem)
        pltpu.async_copy(typ_hbm.at[pl.ds(off, CH)], typ_st.at[st_set], sem)

    def wait_inputs(st_set, sem):
        for ref in (src_st, dst_st, typ_st):
            pltpu.make_async_copy(src_hbm.at[pl.ds(0, CH)], ref.at[st_set],
                                  sem).wait()

    def compute_idx(st_set, gix, six):
        for j in range(CH // 16):
            sl = pl.ds(j * 16, 16)
            sv = src_st[st_set, sl]
            dv = dst_st[st_set, sl]
            tv = typ_st[st_set, sl]
            gi = sv + tv * tstride if tstride else sv
            local = dv - base
            inb = (local >= 0) & (local < QTR)
            si = tv * STRIDE + jnp.where(inb, local, QTR + s)
            gix[pl.ds(j * 16, 16)] = gi
            six[pl.ds(j * 16, 16)] = si

    def wait_cnt(six, semc):
        if with_cnt:
            pltpu.make_async_copy(ones_v, cnt_sp.at[six], semc).wait()

    issue_inputs(0, 0, sem_ia)

    def pair_body(k, carry):
        wait_inputs(0, sem_ia)
        issue_inputs(2 * k + 1, 1, sem_ib)

        @pl.when(k >= 1)
        def _():
            wait_cnt(sixa, sem_ca)
        compute_idx(0, gixa, sixa)
        pltpu.async_copy(table_hbm.at[gixa], rows_v.at[0], sem_g)
        wait_inputs(1, sem_ib)

        @pl.when(k + 1 < npair)
        def _():
            issue_inputs(2 * k + 2, 0, sem_ia)

        @pl.when(k >= 1)
        def _():
            wait_cnt(sixb, sem_cb)
        compute_idx(1, gixb, sixb)
        pltpu.async_copy(table_hbm.at[gixb], rows_v.at[1], sem_gb)
        pltpu.make_async_copy(table_hbm.at[gixa], rows_v.at[0], sem_g).wait()
        pltpu.sync_copy(rows_v.at[0], agg_sp.at[sixa], add=True)
        if with_cnt:
            pltpu.async_copy(ones_v, cnt_sp.at[sixa], sem_ca, add=True)
        pltpu.make_async_copy(table_hbm.at[gixb], rows_v.at[1],
                              sem_gb).wait()
        pltpu.sync_copy(rows_v.at[1], agg_sp.at[sixb], add=True)
        if with_cnt:
            pltpu.async_copy(ones_v, cnt_sp.at[sixb], sem_cb, add=True)
        return carry

    lax.fori_loop(0, npair, pair_body, 0)
    wait_cnt(sixa, sem_ca)
    wait_cnt(sixb, sem_cb)
    plsc.subcore_barrier()

    for b in range(NB):
        r0 = b * STRIDE + s * ROWS_PER_TILE_B
        pltpu.sync_copy(agg_sp.at[pl.ds(r0, ROWS_PER_TILE_B)],
                        agg_hbm.at[b, c, pl.ds(s * ROWS_PER_TILE_B,
                                               ROWS_PER_TILE_B)])
    if with_cnt:
        @pl.when(s == 0)
        def _():
            off = pl.multiple_of(c * SP_ROWS, SP_ROWS)
            for k in range(SP_ROWS // CBUF):
                pltpu.sync_copy(cnt_sp.at[pl.ds(k * CBUF, CBUF)], cbuf_v)
                pltpu.sync_copy(cbuf_v, cnt_hbm.at[pl.ds(off + k * CBUF,
                                                         CBUF)])


def _make_sc_pass(tstride, with_cnt, qpass):
    mesh = plsc.VectorSubcoreMesh(core_axis_name="c", subcore_axis_name="s",
                                  num_cores=NSC, num_subcores=NTILE)
    return pl.kernel(
        functools.partial(_sc_edge_pass, tstride, with_cnt, qpass),
        out_type=(
            jax.ShapeDtypeStruct((NB, NSC, STRIDE, D), jnp.float32),
            jax.ShapeDtypeStruct((NSC * SP_ROWS,), jnp.float32),
        ),
        mesh=mesh,
        scratch_types=[
            pltpu.VMEM_SHARED((SP_ROWS, D), jnp.float32),
            pltpu.VMEM_SHARED((SP_ROWS,), jnp.float32),
            pltpu.VMEM((2, CH, D), jnp.float32),
            pltpu.VMEM((CH,), jnp.int32),
            pltpu.VMEM((CH,), jnp.int32),
            pltpu.VMEM((CH,), jnp.int32),
            pltpu.VMEM((CH,), jnp.int32),
            pltpu.VMEM((2, CH), jnp.int32),
            pltpu.VMEM((2, CH), jnp.int32),
            pltpu.VMEM((2, CH), jnp.int32),
            pltpu.VMEM((CH,), jnp.float32),
            pltpu.VMEM((ZROWS, D), jnp.float32),
            pltpu.VMEM((CBUF,), jnp.float32),
            pltpu.SemaphoreType.DMA,
            pltpu.SemaphoreType.DMA,
            pltpu.SemaphoreType.DMA,
            pltpu.SemaphoreType.DMA,
            pltpu.SemaphoreType.DMA,
            pltpu.SemaphoreType.DMA,
        ],
        name=f"sc_edge_pass_l{1 if tstride else 0}q{qpass}",
    )


def _dense0_body(ue_ref, itf_ref, wu_ref, bu_ref, wi_ref, bi_ref, x_ref):
    xu = jnp.dot(ue_ref[...], wu_ref[...].T,
                 preferred_element_type=jnp.float32) + bu_ref[...]
    xi = jnp.dot(itf_ref[...], wi_ref[...].T,
                 preferred_element_type=jnp.float32) + bi_ref[...]
    x_ref[0:NUM_USERS, :] = xu
    x_ref[NUM_USERS:N, :] = xi


def _layer_body(relu, residual, hin_ref, x_ref, agga_ref, aggb_ref, cnt_ref,
                wr_ref, wn_ref, bias_ref, gamma_ref, beta_ref, out_ref):
    inv = 1.0 / jnp.maximum(cnt_ref[0, 0], 1.0)
    agg = jnp.concatenate([agga_ref[0, 0, :QTR, :], agga_ref[0, 1, :QTR, :],
                           aggb_ref[0, 0, :QTR, :], aggb_ref[0, 1, :QTR, :]],
                          axis=0) * inv[:, None]
    hin = hin_ref[...].reshape(N, D)
    pre = (jnp.dot(hin, wr_ref[0].T, preferred_element_type=jnp.float32)
           + jnp.dot(agg, wn_ref[0].T, preferred_element_type=jnp.float32)
           + bias_ref[0])
    mean = jnp.mean(pre, axis=0)
    var = jnp.mean((pre - mean) ** 2, axis=0)
    h = (pre - mean) / jnp.sqrt(var + EPS) * gamma_ref[0] + beta_ref[0]
    if relu:
        h = jnp.maximum(h, 0.0)
    if residual:
        h = x_ref[...] + h
    out_ref[...] = h.reshape(out_ref.shape)


def _fusion_body(x_ref, stack_ref, qw_ref, qb_ref, kw_ref, kb_ref, fw_ref,
                 fb_ref, rw_ref, rb_ref, out_ref):
    x = x_ref[...]
    q = jnp.dot(x, qw_ref[...].T, preferred_element_type=jnp.float32) \
        + qb_ref[...]
    logits = []
    for b in range(NB):
        kb = jnp.dot(stack_ref[b], kw_ref[b].T,
                     preferred_element_type=jnp.float32) + kb_ref[b]
        logits.append(jnp.sum(q * kb, axis=-1))
    lg = jnp.stack(logits, axis=-1)
    lg = lg - jnp.max(lg, axis=-1, keepdims=True)
    ex = jnp.exp(lg)
    attn = ex / jnp.sum(ex, axis=-1, keepdims=True)
    fused = (attn[:, 0:1] * stack_ref[0] + attn[:, 1:2] * stack_ref[1]
             + attn[:, 2:3] * stack_ref[2])
    fused = jnp.dot(fused, fw_ref[...].T,
                    preferred_element_type=jnp.float32) + fb_ref[...]
    out = jnp.dot(fused, rw_ref[...].T,
                  preferred_element_type=jnp.float32) + rb_ref[...]
    out_ref[...] = jnp.maximum(out, 0.0)


def _layer_call(relu, residual, shared_hin):
    hin_map = (lambda b: (0, 0, 0)) if shared_hin else (lambda b: (b, 0, 0))
    return pl.pallas_call(
        functools.partial(_layer_body, relu, residual),
        grid=(NB,),
        in_specs=[
            pl.BlockSpec((1, N, D), hin_map),
            pl.BlockSpec((N, D), lambda b: (0, 0)),
            pl.BlockSpec((1, NSC, STRIDE, D), lambda b: (b, 0, 0, 0)),
            pl.BlockSpec((1, NSC, STRIDE, D), lambda b: (b, 0, 0, 0)),
            pl.BlockSpec((1, 1, N), lambda b: (b, 0, 0)),
            pl.BlockSpec((1, D, D), lambda b: (b, 0, 0)),
            pl.BlockSpec((1, D, D), lambda b: (b, 0, 0)),
            pl.BlockSpec((1, 1, D), lambda b: (b, 0, 0)),
            pl.BlockSpec((1, 1, D), lambda b: (b, 0, 0)),
            pl.BlockSpec((1, 1, D), lambda b: (b, 0, 0)),
        ],
        out_specs=pl.BlockSpec((1, N, D), lambda b: (b, 0, 0)),
        out_shape=jax.ShapeDtypeStruct((NB, N, D), jnp.float32),
    )


def kernel(edge_index, edge_type, node_ids, item_feats, user_emb,
           user_proj_W, user_proj_b, item_proj_W, item_proj_b,
           sage_Wroot, sage_Wneigh, sage_bias, bn_gamma, bn_beta,
           query_W, query_b, key_W, key_b, fuse_W, fuse_b,
           refine_W, refine_b):
    del node_ids

    ep = ((E + NTILE * 2 * CH - 1) // (NTILE * 2 * CH)) * 2 * CH
    e_pad = ep * NTILE
    pad = e_pad - E
    src = jnp.concatenate([edge_index[0].astype(jnp.int32),
                           jnp.zeros((pad,), jnp.int32)])
    dst = jnp.concatenate([edge_index[1].astype(jnp.int32),
                           jnp.full((pad,), -1, jnp.int32)])
    typ = jnp.concatenate([edge_type.astype(jnp.int32),
                           jnp.zeros((pad,), jnp.int32)])
    zeros_hbm = jnp.zeros((ZROWS, D), jnp.float32)

    x = pl.pallas_call(
        _dense0_body,
        out_shape=jax.ShapeDtypeStruct((N, D), jnp.float32),
    )(user_emb, item_feats, user_proj_W, user_proj_b[None], item_proj_W,
      item_proj_b[None])

    agg0a, cnta = _make_sc_pass(0, True, 0)(x, src, dst, typ, zeros_hbm)
    agg0b, cntb = _make_sc_pass(0, True, 1)(x, src, dst, typ, zeros_hbm)
    ca = cnta.reshape(NSC, NB, STRIDE)[:, :, :QTR]
    cb = cntb.reshape(NSC, NB, STRIDE)[:, :, :QTR]
    cnt = jnp.concatenate([ca.transpose(1, 0, 2), cb.transpose(1, 0, 2)],
                          axis=1).reshape(NB, 1, N)

    h0 = _layer_call(True, False, True)(
        x[None], x, agg0a, agg0b, cnt,
        sage_Wroot[:, 0], sage_Wneigh[:, 0], sage_bias[:, 0][:, None, :],
        bn_gamma[:, 0][:, None, :], bn_beta[:, 0][:, None, :])
    h0_tab = h0.reshape(NB * N, D)

    agg1a, _ = _make_sc_pass(N, False, 0)(h0_tab, src, dst, typ, zeros_hbm)
    agg1b, _ = _make_sc_pass(N, False, 1)(h0_tab, src, dst, typ, zeros_hbm)

    stack = _layer_call(False, True, False)(
        h0, x, agg1a, agg1b, cnt, sage_Wroot[:, 1], sage_Wneigh[:, 1],
        sage_bias[:, 1][:, None, :], bn_gamma[:, 1][:, None, :],
        bn_beta[:, 1][:, None, :])

    BLK = 2000
    final = pl.pallas_call(
        _fusion_body,
        grid=(N // BLK,),
        in_specs=[
            pl.BlockSpec((BLK, D), lambda i: (i, 0)),
            pl.BlockSpec((NB, BLK, D), lambda i: (0, i, 0)),
            pl.BlockSpec((D, D), lambda i: (0, 0)),
            pl.BlockSpec((1, D), lambda i: (0, 0)),
            pl.BlockSpec((NB, D, D), lambda i: (0, 0, 0)),
            pl.BlockSpec((NB, D), lambda i: (0, 0)),
            pl.BlockSpec((D, D), lambda i: (0, 0)),
            pl.BlockSpec((1, D), lambda i: (0, 0)),
            pl.BlockSpec((D, D), lambda i: (0, 0)),
            pl.BlockSpec((1, D), lambda i: (0, 0)),
        ],
        out_specs=pl.BlockSpec((BLK, D), lambda i: (i, 0)),
        out_shape=jax.ShapeDtypeStruct((N, D), jnp.float32),
    )(x, stack, query_W, query_b[None], key_W, key_b, fuse_W, fuse_b[None],
      refine_W, refine_b[None])
    return final


# speedup vs baseline: 1.5888x; 1.0165x over previous
import functools

import jax
import jax.numpy as jnp
from jax import lax
from jax.experimental import pallas as pl
from jax.experimental.pallas import tpu as pltpu
from jax.experimental.pallas import tpu_sc as plsc

N = 10000
D = 128
NB = 3
E = 320000
EPS = 1e-5
NUM_USERS = 5000

QTR = N // 4
STRIDE = 2560
SP_ROWS = NB * STRIDE
NSC = 2
NTILE = 16
CH = 128
ROWS_PER_TILE = SP_ROWS // NTILE
ROWS_PER_TILE_B = STRIDE // NTILE
ZROWS = 96
CBUF = 1280


def _sc_edge_pass(tstride, with_cnt, qpass, table_hbm, src_hbm, dst_hbm,
                  typ_hbm, zeros_hbm, agg_hbm, cnt_hbm, agg_sp, cnt_sp,
                  rows_v, gixa, sixa, gixb, sixb, src_st, dst_st, typ_st,
                  ones_v, zero_v, cbuf_v, sem_ia, sem_ib, sem_g, sem_gb,
                  sem_ca, sem_cb, sem_sa, sem_sb):
    c = lax.axis_index("c")
    s = lax.axis_index("s")
    base = (2 * qpass + c) * QTR
    ep = src_hbm.shape[0] // NTILE
    tbase = s * ep
    npair = ep // (2 * CH)

    pltpu.sync_copy(zeros_hbm, zero_v)
    ones16 = jnp.full((16,), 1.0, dtype=jnp.float32)
    zeros16 = jnp.zeros((16,), dtype=jnp.float32)
    for j in range(CH // 16):
        ones_v[pl.ds(j * 16, 16)] = ones16
    for j in range(CBUF // 16):
        cbuf_v[pl.ds(j * 16, 16)] = zeros16
    zrow = s * ROWS_PER_TILE
    for z in range(ROWS_PER_TILE // ZROWS):
        pltpu.sync_copy(zero_v, agg_sp.at[pl.ds(zrow + z * ZROWS, ZROWS)])
    if with_cnt:
        pltpu.sync_copy(cbuf_v.at[pl.ds(0, ROWS_PER_TILE)],
                        cnt_sp.at[pl.ds(zrow, ROWS_PER_TILE)])
    plsc.subcore_barrier()

    def issue_inputs(g, st_set, sem):
        off = pl.multiple_of(tbase + g * CH, CH)
        pltpu.async_copy(src_hbm.at[pl.ds(off, CH)], src_st.at[st_set], sem)
        pltpu.async_copy(dst_hbm.at[pl.ds(off, CH)], dst_st.at[st_set], sem)
        pltpu.async_copy(typ_hbm.at[pl.ds(off, CH)], typ_st.at[st_set], sem)

    def wait_inputs(st_set, sem):
        for ref in (src_st, dst_st, typ_st):
            pltpu.make_async_copy(src_hbm.at[pl.ds(0, CH)], ref.at[st_set],
                                  sem).wait()

    def compute_idx(st_set, gix, six):
        for j in range(CH // 16):
            sl = pl.ds(j * 16, 16)
            sv = src_st[st_set, sl]
            dv = dst_st[st_set, sl]
            tv = typ_st[st_set, sl]
            gi = sv + tv * tstride if tstride else sv
            local = dv - base
            inb = (local >= 0) & (local < QTR)
            si = tv * STRIDE + jnp.where(inb, local, QTR + s)
            gix[pl.ds(j * 16, 16)] = gi
            six[pl.ds(j * 16, 16)] = si

    def wait_cnt(six, semc):
        if with_cnt:
            pltpu.make_async_copy(ones_v, cnt_sp.at[six], semc).wait()

    issue_inputs(0, 0, sem_ia)

    def pair_body(k, carry):
        wait_inputs(0, sem_ia)
        issue_inputs(2 * k + 1, 1, sem_ib)

        @pl.when(k >= 1)
        def _():
            wait_cnt(sixa, sem_ca)
            pltpu.make_async_copy(rows_v.at[0], agg_sp.at[sixa],
                                  sem_sa).wait()
        compute_idx(0, gixa, sixa)
        pltpu.async_copy(table_hbm.at[gixa], rows_v.at[0], sem_g)
        wait_inputs(1, sem_ib)

        @pl.when(k + 1 < npair)
        def _():
            issue_inputs(2 * k + 2, 0, sem_ia)

        @pl.when(k >= 1)
        def _():
            wait_cnt(sixb, sem_cb)
            pltpu.make_async_copy(rows_v.at[1], agg_sp.at[sixb],
                                  sem_sb).wait()
        compute_idx(1, gixb, sixb)
        pltpu.async_copy(table_hbm.at[gixb], rows_v.at[1], sem_gb)
        pltpu.make_async_copy(table_hbm.at[gixa], rows_v.at[0], sem_g).wait()
        pltpu.async_copy(rows_v.at[0], agg_sp.at[sixa], sem_sa, add=True)
        if with_cnt:
            pltpu.async_copy(ones_v, cnt_sp.at[sixa], sem_ca, add=True)
        pltpu.make_async_copy(table_hbm.at[gixb], rows_v.at[1],
                              sem_gb).wait()
        pltpu.async_copy(rows_v.at[1], agg_sp.at[sixb], sem_sb, add=True)
        if with_cnt:
            pltpu.async_copy(ones_v, cnt_sp.at[sixb], sem_cb, add=True)
        return carry

    lax.fori_loop(0, npair, pair_body, 0)
    wait_cnt(sixa, sem_ca)
    wait_cnt(sixb, sem_cb)
    pltpu.make_async_copy(rows_v.at[0], agg_sp.at[sixa], sem_sa).wait()
    pltpu.make_async_copy(rows_v.at[1], agg_sp.at[sixb], sem_sb).wait()
    plsc.subcore_barrier()

    for b in range(NB):
        r0 = b * STRIDE + s * ROWS_PER_TILE_B
        pltpu.sync_copy(agg_sp.at[pl.ds(r0, ROWS_PER_TILE_B)],
                        agg_hbm.at[b, c, pl.ds(s * ROWS_PER_TILE_B,
                                               ROWS_PER_TILE_B)])
    if with_cnt:
        @pl.when(s == 0)
        def _():
            off = pl.multiple_of(c * SP_ROWS, SP_ROWS)
            for k in range(SP_ROWS // CBUF):
                pltpu.sync_copy(cnt_sp.at[pl.ds(k * CBUF, CBUF)], cbuf_v)
                pltpu.sync_copy(cbuf_v, cnt_hbm.at[pl.ds(off + k * CBUF,
                                                         CBUF)])


def _make_sc_pass(tstride, with_cnt, qpass):
    mesh = plsc.VectorSubcoreMesh(core_axis_name="c", subcore_axis_name="s",
                                  num_cores=NSC, num_subcores=NTILE)
    return pl.kernel(
        functools.partial(_sc_edge_pass, tstride, with_cnt, qpass),
        out_type=(
            jax.ShapeDtypeStruct((NB, NSC, STRIDE, D), jnp.float32),
            jax.ShapeDtypeStruct((NSC * SP_ROWS,), jnp.float32),
        ),
        mesh=mesh,
        scratch_types=[
            pltpu.VMEM_SHARED((SP_ROWS, D), jnp.float32),
            pltpu.VMEM_SHARED((SP_ROWS,), jnp.float32),
            pltpu.VMEM((2, CH, D), jnp.float32),
            pltpu.VMEM((CH,), jnp.int32),
            pltpu.VMEM((CH,), jnp.int32),
            pltpu.VMEM((CH,), jnp.int32),
            pltpu.VMEM((CH,), jnp.int32),
            pltpu.VMEM((2, CH), jnp.int32),
            pltpu.VMEM((2, CH), jnp.int32),
            pltpu.VMEM((2, CH), jnp.int32),
            pltpu.VMEM((CH,), jnp.float32),
            pltpu.VMEM((ZROWS, D), jnp.float32),
            pltpu.VMEM((CBUF,), jnp.float32),
            pltpu.SemaphoreType.DMA,
            pltpu.SemaphoreType.DMA,
            pltpu.SemaphoreType.DMA,
            pltpu.SemaphoreType.DMA,
            pltpu.SemaphoreType.DMA,
            pltpu.SemaphoreType.DMA,
            pltpu.SemaphoreType.DMA,
            pltpu.SemaphoreType.DMA,
        ],
        name=f"sc_edge_pass_l{1 if tstride else 0}q{qpass}",
    )


def _dense0_body(ue_ref, itf_ref, wu_ref, bu_ref, wi_ref, bi_ref, x_ref):
    xu = jnp.dot(ue_ref[...], wu_ref[...].T,
                 preferred_element_type=jnp.float32) + bu_ref[...]
    xi = jnp.dot(itf_ref[...], wi_ref[...].T,
                 preferred_element_type=jnp.float32) + bi_ref[...]
    x_ref[0:NUM_USERS, :] = xu
    x_ref[NUM_USERS:N, :] = xi


def _layer_body(relu, residual, hin_ref, x_ref, agga_ref, aggb_ref, cnt_ref,
                wr_ref, wn_ref, bias_ref, gamma_ref, beta_ref, out_ref):
    inv = 1.0 / jnp.maximum(cnt_ref[0, 0], 1.0)
    agg = jnp.concatenate([agga_ref[0, 0, :QTR, :], agga_ref[0, 1, :QTR, :],
                           aggb_ref[0, 0, :QTR, :], aggb_ref[0, 1, :QTR, :]],
                          axis=0) * inv[:, None]
    hin = hin_ref[...].reshape(N, D)
    pre = (jnp.dot(hin, wr_ref[0].T, preferred_element_type=jnp.float32)
           + jnp.dot(agg, wn_ref[0].T, preferred_element_type=jnp.float32)
           + bias_ref[0])
    mean = jnp.mean(pre, axis=0)
    var = jnp.mean((pre - mean) ** 2, axis=0)
    h = (pre - mean) / jnp.sqrt(var + EPS) * gamma_ref[0] + beta_ref[0]
    if relu:
        h = jnp.maximum(h, 0.0)
    if residual:
        h = x_ref[...] + h
    out_ref[...] = h.reshape(out_ref.shape)


def _fusion_body(x_ref, stack_ref, qw_ref, qb_ref, kw_ref, kb_ref, fw_ref,
                 fb_ref, rw_ref, rb_ref, out_ref):
    x = x_ref[...]
    q = jnp.dot(x, qw_ref[...].T, preferred_element_type=jnp.float32) \
        + qb_ref[...]
    logits = []
    for b in range(NB):
        kb = jnp.dot(stack_ref[b], kw_ref[b].T,
                     preferred_element_type=jnp.float32) + kb_ref[b]
        logits.append(jnp.sum(q * kb, axis=-1))
    lg = jnp.stack(logits, axis=-1)
    lg = lg - jnp.max(lg, axis=-1, keepdims=True)
    ex = jnp.exp(lg)
    attn = ex / jnp.sum(ex, axis=-1, keepdims=True)
    fused = (attn[:, 0:1] * stack_ref[0] + attn[:, 1:2] * stack_ref[1]
             + attn[:, 2:3] * stack_ref[2])
    fused = jnp.dot(fused, fw_ref[...].T,
                    preferred_element_type=jnp.float32) + fb_ref[...]
    out = jnp.dot(fused, rw_ref[...].T,
                  preferred_element_type=jnp.float32) + rb_ref[...]
    out_ref[...] = jnp.maximum(out, 0.0)


def _layer_call(relu, residual, shared_hin):
    hin_map = (lambda b: (0, 0, 0)) if shared_hin else (lambda b: (b, 0, 0))
    return pl.pallas_call(
        functools.partial(_layer_body, relu, residual),
        grid=(NB,),
        in_specs=[
            pl.BlockSpec((1, N, D), hin_map),
            pl.BlockSpec((N, D), lambda b: (0, 0)),
            pl.BlockSpec((1, NSC, STRIDE, D), lambda b: (b, 0, 0, 0)),
            pl.BlockSpec((1, NSC, STRIDE, D), lambda b: (b, 0, 0, 0)),
            pl.BlockSpec((1, 1, N), lambda b: (b, 0, 0)),
            pl.BlockSpec((1, D, D), lambda b: (b, 0, 0)),
            pl.BlockSpec((1, D, D), lambda b: (b, 0, 0)),
            pl.BlockSpec((1, 1, D), lambda b: (b, 0, 0)),
            pl.BlockSpec((1, 1, D), lambda b: (b, 0, 0)),
            pl.BlockSpec((1, 1, D), lambda b: (b, 0, 0)),
        ],
        out_specs=pl.BlockSpec((1, N, D), lambda b: (b, 0, 0)),
        out_shape=jax.ShapeDtypeStruct((NB, N, D), jnp.float32),
    )


def kernel(edge_index, edge_type, node_ids, item_feats, user_emb,
           user_proj_W, user_proj_b, item_proj_W, item_proj_b,
           sage_Wroot, sage_Wneigh, sage_bias, bn_gamma, bn_beta,
           query_W, query_b, key_W, key_b, fuse_W, fuse_b,
           refine_W, refine_b):
    del node_ids

    ep = ((E + NTILE * 2 * CH - 1) // (NTILE * 2 * CH)) * 2 * CH
    e_pad = ep * NTILE
    pad = e_pad - E
    src = jnp.concatenate([edge_index[0].astype(jnp.int32),
                           jnp.zeros((pad,), jnp.int32)])
    dst = jnp.concatenate([edge_index[1].astype(jnp.int32),
                           jnp.full((pad,), -1, jnp.int32)])
    typ = jnp.concatenate([edge_type.astype(jnp.int32),
                           jnp.zeros((pad,), jnp.int32)])
    zeros_hbm = jnp.zeros((ZROWS, D), jnp.float32)

    x = pl.pallas_call(
        _dense0_body,
        out_shape=jax.ShapeDtypeStruct((N, D), jnp.float32),
    )(user_emb, item_feats, user_proj_W, user_proj_b[None], item_proj_W,
      item_proj_b[None])

    agg0a, cnta = _make_sc_pass(0, True, 0)(x, src, dst, typ, zeros_hbm)
    agg0b, cntb = _make_sc_pass(0, True, 1)(x, src, dst, typ, zeros_hbm)
    ca = cnta.reshape(NSC, NB, STRIDE)[:, :, :QTR]
    cb = cntb.reshape(NSC, NB, STRIDE)[:, :, :QTR]
    cnt = jnp.concatenate([ca.transpose(1, 0, 2), cb.transpose(1, 0, 2)],
                          axis=1).reshape(NB, 1, N)

    h0 = _layer_call(True, False, True)(
        x[None], x, agg0a, agg0b, cnt,
        sage_Wroot[:, 0], sage_Wneigh[:, 0], sage_bias[:, 0][:, None, :],
        bn_gamma[:, 0][:, None, :], bn_beta[:, 0][:, None, :])
    h0_tab = h0.reshape(NB * N, D)

    agg1a, _ = _make_sc_pass(N, False, 0)(h0_tab, src, dst, typ, zeros_hbm)
    agg1b, _ = _make_sc_pass(N, False, 1)(h0_tab, src, dst, typ, zeros_hbm)

    stack = _layer_call(False, True, False)(
        h0, x, agg1a, agg1b, cnt, sage_Wroot[:, 1], sage_Wneigh[:, 1],
        sage_bias[:, 1][:, None, :], bn_gamma[:, 1][:, None, :],
        bn_beta[:, 1][:, None, :])

    BLK = 2000
    final = pl.pallas_call(
        _fusion_body,
        grid=(N // BLK,),
        in_specs=[
            pl.BlockSpec((BLK, D), lambda i: (i, 0)),
            pl.BlockSpec((NB, BLK, D), lambda i: (0, i, 0)),
            pl.BlockSpec((D, D), lambda i: (0, 0)),
            pl.BlockSpec((1, D), lambda i: (0, 0)),
            pl.BlockSpec((NB, D, D), lambda i: (0, 0, 0)),
            pl.BlockSpec((NB, D), lambda i: (0, 0)),
            pl.BlockSpec((D, D), lambda i: (0, 0)),
            pl.BlockSpec((1, D), lambda i: (0, 0)),
            pl.BlockSpec((D, D), lambda i: (0, 0)),
            pl.BlockSpec((1, D), lambda i: (0, 0)),
        ],
        out_specs=pl.BlockSpec((BLK, D), lambda i: (i, 0)),
        out_shape=jax.ShapeDtypeStruct((N, D), jnp.float32),
    )(x, stack, query_W, query_b[None], key_W, key_b, fuse_W, fuse_b[None],
      refine_W, refine_b[None])
    return final
